# baseline (speedup 1.0000x reference)
import jax
import jax.numpy as jnp
from jax.experimental import pallas as pl
from jax.experimental.pallas import tpu as pltpu

FUSE_K = 20
TN = 1024
ROWS = TN // 8
SLOTS = 8 * FUSE_K
NCHUNK = 4
RPC = ROWS // NCHUNK
LPC = TN * FUSE_K // NCHUNK


def _pool_kernel(r_ref, ct_ref, feat_ref, out_ref, idx_smem, sems,
                 sc_a, sc_b, sc_c, sc_d):
    for q in range(NCHUNK):
        pltpu.make_async_copy(r_ref.at[pl.ds(q * LPC, LPC)],
                              idx_smem.at[pl.ds(q * LPC, LPC)],
                              sems.at[q]).start()

    lane = jax.lax.broadcasted_iota(jnp.int32, (8, 128), 1)
    sub = jax.lax.broadcasted_iota(jnp.int32, (8, 128), 0)
    msk = (lane >> 4) == sub

    def one_row(g, scratch):
        base = g * SLOTS
        for j in range(8):
            for k in range(FUSE_K):
                b = idx_smem[base + j * FUSE_K + k]
                scratch[k * 8 + j:k * 8 + j + 1, :] = feat_ref[b]
        ctile = ct_ref[g]
        accs = [jnp.zeros((8, 128), jnp.float32) for _ in range(4)]
        for k in range(FUSE_K):
            chunk = scratch[k * 8:(k + 1) * 8, :]
            ckb = jnp.broadcast_to(ctile[:, k:k + 1], (8, 128))
            idxv = (lane + ckb) & 127
            accs[k & 3] = accs[k & 3] + jnp.take_along_axis(chunk, idxv, axis=1)
        acc = (accs[0] + accs[1]) + (accs[2] + accs[3])
        tot = jnp.sum(jnp.where(msk, acc, 0.0), axis=0, keepdims=True)
        out_ref[g] = tot * (1.0 / FUSE_K)

    def body(gg, carry):
        one_row(4 * gg, sc_a)
        one_row(4 * gg + 1, sc_b)
        one_row(4 * gg + 2, sc_c)
        one_row(4 * gg + 3, sc_d)
        return carry

    for q in range(NCHUNK):
        pltpu.make_async_copy(r_ref.at[pl.ds(q * LPC, LPC)],
                              idx_smem.at[pl.ds(q * LPC, LPC)],
                              sems.at[q]).wait()
        jax.lax.fori_loop(q * RPC // 4, (q + 1) * RPC // 4, body, 0)


def kernel(invoxel_xyz, invoxel_map, src_feat, voxel_center):
    del invoxel_xyz, voxel_center

    idx = invoxel_map[:, :FUSE_K].astype(jnp.int32)
    idx = jnp.where(idx == 0, idx[:, :1], idx)

    N = idx.shape[0]
    M, C = src_feat.shape
    assert C == 16, "feature packing below assumes C == 16"

    n_tiles = pl.cdiv(N, TN)
    n_pad = n_tiles * TN
    idx_p = jnp.zeros((n_pad, FUSE_K), jnp.int32).at[:N].set(idx)

    r_flat = (idx_p >> 3).reshape(-1)
    s = (idx_p & 7).reshape(n_pad // 8, 8, FUSE_K)
    j_arr = jnp.arange(8, dtype=jnp.int32)[None, :, None]
    ct = ((s - j_arr) * 16) & 127

    m_pad = ((M + 7) // 8) * 8
    feat = src_feat.astype(jnp.float32)
    if m_pad != M:
        feat = jnp.zeros((m_pad, C), jnp.float32).at[:M].set(feat)
    feat3d = feat.reshape(m_pad // 8, 1, 128)

    out = pl.pallas_call(
        _pool_kernel,
        out_shape=jax.ShapeDtypeStruct((n_tiles * ROWS, 1, 128), jnp.float32),
        grid=(n_tiles,),
        in_specs=[
            pl.BlockSpec((TN * FUSE_K,), lambda n: (n,)),
            pl.BlockSpec((ROWS, 8, FUSE_K), lambda n: (n, 0, 0)),
            pl.BlockSpec((m_pad // 8, 1, 128), lambda n: (0, 0, 0)),
        ],
        out_specs=pl.BlockSpec((ROWS, 1, 128), lambda n: (n, 0, 0)),
        scratch_shapes=[
            pltpu.SMEM((TN * FUSE_K,), jnp.int32),
            pltpu.SemaphoreType.DMA((NCHUNK,)),
            pltpu.VMEM((SLOTS, 128), jnp.float32),
            pltpu.VMEM((SLOTS, 128), jnp.float32),
            pltpu.VMEM((SLOTS, 128), jnp.float32),
            pltpu.VMEM((SLOTS, 128), jnp.float32),
        ],
        compiler_params=pltpu.CompilerParams(
            dimension_semantics=("arbitrary",),
            vmem_limit_bytes=56 * 1024 * 1024,
        ),
    )(r_flat, ct, feat3d)

    return out.reshape(n_tiles * TN, C)[:N].astype(src_feat.dtype)


# speedup vs baseline: 2.2857x; 2.2857x over previous
import jax
import jax.numpy as jnp
from jax.experimental import pallas as pl
from jax.experimental.pallas import tpu as pltpu

FUSE_K = 20
TN = 1024
ROWS = TN // 8
SLOTS = 8 * FUSE_K
NCHUNK = 4
RPC = ROWS // NCHUNK
LPC = TN * FUSE_K // NCHUNK


def _pool_kernel(r_ref, ct_ref, feat_ref, out_ref, idx_smem, sems, sc_a, sc_b):
    for q in range(NCHUNK):
        pltpu.make_async_copy(r_ref.at[pl.ds(q * LPC, LPC)],
                              idx_smem.at[pl.ds(q * LPC, LPC)],
                              sems.at[q]).start()

    lane = jax.lax.broadcasted_iota(jnp.int32, (8, 128), 1)
    sub = jax.lax.broadcasted_iota(jnp.int32, (8, 128), 0)
    msk = (lane >> 4) == sub

    def one_row(g, scratch):
        base = g * SLOTS
        for j in range(8):
            for k in range(FUSE_K):
                b = idx_smem[base + j * FUSE_K + k]
                scratch[k * 8 + j:k * 8 + j + 1, :] = feat_ref[b]
        ctile = ct_ref[g]
        accs = [jnp.zeros((8, 128), jnp.float32) for _ in range(4)]
        for k in range(FUSE_K):
            chunk = scratch[k * 8:(k + 1) * 8, :]
            ckb = jnp.broadcast_to(ctile[:, k:k + 1], (8, 128))
            idxv = (lane + ckb) & 127
            accs[k & 3] = accs[k & 3] + jnp.take_along_axis(chunk, idxv, axis=1)
        acc = (accs[0] + accs[1]) + (accs[2] + accs[3])
        tot = jnp.sum(jnp.where(msk, acc, 0.0), axis=0, keepdims=True)
        out_ref[g] = tot * (1.0 / FUSE_K)

    def body(gg, carry):
        one_row(2 * gg, sc_a)
        one_row(2 * gg + 1, sc_b)
        return carry

    for q in range(NCHUNK):
        pltpu.make_async_copy(r_ref.at[pl.ds(q * LPC, LPC)],
                              idx_smem.at[pl.ds(q * LPC, LPC)],
                              sems.at[q]).wait()
        jax.lax.fori_loop(q * RPC // 2, (q + 1) * RPC // 2, body, 0)


def kernel(invoxel_xyz, invoxel_map, src_feat, voxel_center):
    del invoxel_xyz, voxel_center

    idx = invoxel_map[:, :FUSE_K].astype(jnp.int32)
    idx = jnp.where(idx == 0, idx[:, :1], idx)

    N = idx.shape[0]
    M, C = src_feat.shape
    assert C == 16, "feature packing below assumes C == 16"

    n_tiles = pl.cdiv(N, TN)
    n_pad = n_tiles * TN
    idx_p = jnp.zeros((n_pad, FUSE_K), jnp.int32).at[:N].set(idx)

    r_flat = (idx_p >> 3).reshape(-1)
    s = (idx_p & 7).reshape(n_pad // 8, 8, FUSE_K)
    j_arr = jnp.arange(8, dtype=jnp.int32)[None, :, None]
    ct = ((s - j_arr) * 16) & 127

    m_pad = ((M + 7) // 8) * 8
    feat = src_feat.astype(jnp.float32)
    if m_pad != M:
        feat = jnp.zeros((m_pad, C), jnp.float32).at[:M].set(feat)
    feat3d = feat.reshape(m_pad // 8, 1, 128)

    out = pl.pallas_call(
        _pool_kernel,
        out_shape=jax.ShapeDtypeStruct((n_tiles * ROWS, 1, 128), jnp.float32),
        grid=(n_tiles,),
        in_specs=[
            pl.BlockSpec((TN * FUSE_K,), lambda n: (n,)),
            pl.BlockSpec((ROWS, 8, FUSE_K), lambda n: (n, 0, 0)),
            pl.BlockSpec((m_pad // 8, 1, 128), lambda n: (0, 0, 0)),
        ],
        out_specs=pl.BlockSpec((ROWS, 1, 128), lambda n: (n, 0, 0)),
        scratch_shapes=[
            pltpu.SMEM((TN * FUSE_K,), jnp.int32),
            pltpu.SemaphoreType.DMA((NCHUNK,)),
            pltpu.VMEM((SLOTS, 128), jnp.float32),
            pltpu.VMEM((SLOTS, 128), jnp.float32),
        ],
        compiler_params=pltpu.CompilerParams(
            dimension_semantics=("arbitrary",),
            vmem_limit_bytes=56 * 1024 * 1024,
        ),
    )(r_flat, ct, feat3d)

    return out.reshape(n_tiles * TN, C)[:N].astype(src_feat.dtype)
